# Initial kernel scaffold; baseline (speedup 1.0000x reference)
import jax
import jax.numpy as jnp
from jax.experimental import pallas as pl


def kernel(x, edge_index, batch, W1, b1, g1, beta1, W2, b2, g2, beta2):
    raise NotImplementedError("write your pallas kernel here")


# speedup vs baseline: 18.2489x; 18.2489x over previous
import functools

import jax
import jax.numpy as jnp
from jax import lax
from jax.experimental import pallas as pl
from jax.experimental.pallas import tpu as pltpu
from jax.experimental.pallas import tpu_sc as plsc

N = 10000
E = 320000
D = 128
H = 64
G = 64
EPS = 1e-5

NPAD = 10240
EPAD = 327680
CH = 128
PER_TILE = EPAD // 32
NCHUNK = PER_TILE // CH
ROWS_PER_TILE = NPAD // 16

BN = 1000
GRID = N // BN

_mesh = plsc.VectorSubcoreMesh(core_axis_name="c", subcore_axis_name="s")


@functools.partial(
    pl.kernel,
    out_type=jax.ShapeDtypeStruct((2 * NPAD,), jnp.float32),
    mesh=_mesh,
    compiler_params=pltpu.CompilerParams(use_tc_tiling_on_sc=False),
    scratch_types=[
        pltpu.VMEM((CH,), jnp.int32),
        pltpu.VMEM((CH,), jnp.float32),
        pltpu.VMEM((ROWS_PER_TILE,), jnp.float32),
        pltpu.VMEM_SHARED((NPAD,), jnp.float32),
        pltpu.SemaphoreType.DMA,
    ],
)
def _deg_kernel(dst_hbm, out_hbm, idx_v, ones_v, zbuf_v, cnt_sp, sem):
    c = lax.axis_index("c")
    s = lax.axis_index("s")
    gid = c * 16 + s

    def fill(i, _):
        zbuf_v[pl.ds(i * 16, 16)] = jnp.zeros((16,), jnp.float32)
        return 0

    lax.fori_loop(0, ROWS_PER_TILE // 16, fill, 0)

    def fill1(i, _):
        ones_v[pl.ds(i * 16, 16)] = jnp.ones((16,), jnp.float32)
        return 0

    lax.fori_loop(0, CH // 16, fill1, 0)

    pltpu.sync_copy(zbuf_v, cnt_sp.at[pl.ds(s * ROWS_PER_TILE, ROWS_PER_TILE)])
    plsc.subcore_barrier()

    def body(k, _):
        base = gid * PER_TILE + k * CH
        pltpu.sync_copy(dst_hbm.at[pl.ds(base, CH)], idx_v)
        pltpu.sync_copy(ones_v, cnt_sp.at[idx_v], add=True)
        return 0

    lax.fori_loop(0, NCHUNK, body, 0)
    plsc.subcore_barrier()
    pltpu.sync_copy(
        cnt_sp.at[pl.ds(s * ROWS_PER_TILE, ROWS_PER_TILE)],
        out_hbm.at[pl.ds(c * NPAD + s * ROWS_PER_TILE, ROWS_PER_TILE)],
    )


@functools.partial(
    pl.kernel,
    out_type=jax.ShapeDtypeStruct((2 * NPAD, H), jnp.float32),
    mesh=_mesh,
    compiler_params=pltpu.CompilerParams(use_tc_tiling_on_sc=False),
    scratch_types=[
        pltpu.VMEM((CH,), jnp.int32),
        pltpu.VMEM((CH,), jnp.int32),
        pltpu.VMEM((CH, H), jnp.float32),
        pltpu.VMEM_SHARED((NPAD, H), jnp.float32),
        pltpu.SemaphoreType.DMA,
    ],
)
def _agg_kernel(hp_hbm, src_hbm, dst_hbm, out_hbm, sidx_v, didx_v, rows_v, acc_sp, sem):
    c = lax.axis_index("c")
    s = lax.axis_index("s")
    gid = c * 16 + s

    def fill(t, _):
        rows_v[t // 4, pl.ds((t % 4) * 16, 16)] = jnp.zeros((16,), jnp.float32)
        return 0

    lax.fori_loop(0, CH * (H // 16), fill, 0)

    def zc(k, _):
        pltpu.sync_copy(rows_v, acc_sp.at[pl.ds(s * ROWS_PER_TILE + k * CH, CH), :])
        return 0

    lax.fori_loop(0, ROWS_PER_TILE // CH, zc, 0)
    plsc.subcore_barrier()

    def body(k, _):
        base = gid * PER_TILE + k * CH
        pltpu.sync_copy(src_hbm.at[pl.ds(base, CH)], sidx_v)
        pltpu.sync_copy(dst_hbm.at[pl.ds(base, CH)], didx_v)
        pltpu.async_copy(hp_hbm.at[sidx_v], rows_v, sem).wait()
        pltpu.sync_copy(rows_v, acc_sp.at[didx_v], add=True)
        return 0

    lax.fori_loop(0, NCHUNK, body, 0)
    plsc.subcore_barrier()
    pltpu.sync_copy(
        acc_sp.at[pl.ds(s * ROWS_PER_TILE, ROWS_PER_TILE), :],
        out_hbm.at[pl.ds(c * NPAD + s * ROWS_PER_TILE, ROWS_PER_TILE), :],
    )


def _dense1_body(cnt_ref, x_ref, w1_ref, h_ref, hp_ref):
    cnt2 = cnt_ref[...]
    deg = cnt2[0] + cnt2[1] + 1.0
    dinv = lax.rsqrt(deg)
    h = jnp.dot(x_ref[...], w1_ref[...], preferred_element_type=jnp.float32)
    h_ref[...] = h
    hp_ref[...] = h * dinv


def _dense2_body(parts_ref, h1_ref, cnt_ref, b1_ref, g1_ref, be1_ref, w2_ref,
                 h2_ref, h2p_ref):
    p = parts_ref[...]
    agg = p[0] + p[1]
    cnt2 = cnt_ref[...]
    deg = cnt2[0] + cnt2[1] + 1.0
    dinv = lax.rsqrt(deg)
    z = dinv * agg + (1.0 / deg) * h1_ref[...] + b1_ref[...]
    z = z * (g1_ref[...] / jnp.sqrt(1.0 + EPS)) + be1_ref[...]
    z = jnp.maximum(z, 0.0)
    h2 = jnp.dot(z, w2_ref[...], preferred_element_type=jnp.float32)
    h2_ref[...] = h2
    h2p_ref[...] = h2 * lax.rsqrt(deg)


def _pool_body(parts_ref, h2_ref, cnt_ref, b2_ref, g2_ref, be2_ref, batch_ref,
               out_ref, sums, cnts):
    i = pl.program_id(0)

    @pl.when(i == 0)
    def _():
        sums[...] = jnp.zeros_like(sums)
        cnts[...] = jnp.zeros_like(cnts)

    p = parts_ref[...]
    agg = p[0] + p[1]
    cnt2 = cnt_ref[...]
    deg = cnt2[0] + cnt2[1] + 1.0
    dinv = lax.rsqrt(deg)
    z = dinv * agg + (1.0 / deg) * h2_ref[...] + b2_ref[...]
    z = z * (g2_ref[...] / jnp.sqrt(1.0 + EPS)) + be2_ref[...]
    z = jnp.maximum(z, 0.0)

    b = batch_ref[...]
    onehot = (b == lax.broadcasted_iota(jnp.int32, (BN, G), 1)).astype(jnp.float32)
    sums[...] += lax.dot_general(onehot, z, (((0,), (0,)), ((), ())),
                                 preferred_element_type=jnp.float32)
    cnts[...] += lax.dot_general(onehot, jnp.ones((BN, 1), jnp.float32),
                                 (((0,), (0,)), ((), ())),
                                 preferred_element_type=jnp.float32)
    out_ref[...] = sums[...] / jnp.maximum(cnts[...], 1.0)


def kernel(x, edge_index, batch, W1, b1, g1, beta1, W2, b2, g2, beta2):
    src = edge_index[0].astype(jnp.int32)
    dst = edge_index[1].astype(jnp.int32)
    npad_extra = jnp.arange(EPAD - E, dtype=jnp.int32)
    src_p = jnp.concatenate([src, npad_extra % 256])
    dst_p = jnp.concatenate([dst, N + npad_extra % (NPAD - N)])

    cnt_parts = _deg_kernel(dst_p)
    cnt = cnt_parts.reshape(2, NPAD)[:, :N].reshape(2, N, 1)

    cnt_spec = pl.BlockSpec((2, BN, 1), lambda i: (0, i, 0))
    row_spec = pl.BlockSpec((BN, H), lambda i: (i, 0))
    vec_spec = pl.BlockSpec((1, H), lambda i: (0, 0))
    parts_spec = pl.BlockSpec((2, BN, H), lambda i: (0, i, 0))

    h1, h1p = pl.pallas_call(
        _dense1_body,
        grid=(GRID,),
        in_specs=[cnt_spec,
                  pl.BlockSpec((BN, D), lambda i: (i, 0)),
                  pl.BlockSpec((D, H), lambda i: (0, 0))],
        out_specs=[row_spec, row_spec],
        out_shape=[jax.ShapeDtypeStruct((N, H), jnp.float32),
                   jax.ShapeDtypeStruct((N, H), jnp.float32)],
    )(cnt, x, W1)

    agg1 = _agg_kernel(h1p, src_p, dst_p)
    agg1 = agg1.reshape(2, NPAD, H)[:, :N]

    h2, h2p = pl.pallas_call(
        _dense2_body,
        grid=(GRID,),
        in_specs=[parts_spec, row_spec, cnt_spec, vec_spec, vec_spec, vec_spec,
                  pl.BlockSpec((H, H), lambda i: (0, 0))],
        out_specs=[row_spec, row_spec],
        out_shape=[jax.ShapeDtypeStruct((N, H), jnp.float32),
                   jax.ShapeDtypeStruct((N, H), jnp.float32)],
    )(agg1, h1, cnt, b1.reshape(1, H), g1.reshape(1, H), beta1.reshape(1, H), W2)

    agg2 = _agg_kernel(h2p, src_p, dst_p)
    agg2 = agg2.reshape(2, NPAD, H)[:, :N]

    emb = pl.pallas_call(
        _pool_body,
        grid=(GRID,),
        in_specs=[parts_spec, row_spec, cnt_spec, vec_spec, vec_spec, vec_spec,
                  pl.BlockSpec((BN, 1), lambda i: (i, 0))],
        out_specs=pl.BlockSpec((G, H), lambda i: (0, 0)),
        out_shape=jax.ShapeDtypeStruct((G, H), jnp.float32),
        scratch_shapes=[pltpu.VMEM((G, H), jnp.float32),
                        pltpu.VMEM((G, 1), jnp.float32)],
    )(agg2, h2, cnt, b2.reshape(1, H), g2.reshape(1, H), beta2.reshape(1, H),
      batch.astype(jnp.int32).reshape(N, 1))
    return emb


# speedup vs baseline: 37.3738x; 2.0480x over previous
import functools

import jax
import jax.numpy as jnp
from jax import lax
from jax.experimental import pallas as pl
from jax.experimental.pallas import tpu as pltpu
from jax.experimental.pallas import tpu_sc as plsc

N = 10000
E = 320000
D = 128
H = 64
G = 64
EPS = 1e-5

NPAD = 10240
EPAD = 327680
CH = 128
PER_TILE = EPAD // 32
NCHUNK = PER_TILE // CH
ROWS_PER_TILE = NPAD // 16

BN = 1000
GRID = N // BN

_mesh = plsc.VectorSubcoreMesh(core_axis_name="c", subcore_axis_name="s")


@functools.partial(
    pl.kernel,
    out_type=jax.ShapeDtypeStruct((2 * NPAD,), jnp.float32),
    mesh=_mesh,
    compiler_params=pltpu.CompilerParams(use_tc_tiling_on_sc=False),
    scratch_types=[
        pltpu.VMEM((NCHUNK, 1, CH), jnp.int32),
        pltpu.VMEM((CH,), jnp.float32),
        pltpu.VMEM((ROWS_PER_TILE,), jnp.float32),
        pltpu.VMEM_SHARED((NPAD,), jnp.float32),
    ],
)
def _deg_kernel(dst_hbm, out_hbm, didx, ones_v, zbuf_v, cnt_sp):
    c = lax.axis_index("c")
    s = lax.axis_index("s")
    gid = c * 16 + s

    def fill(i, _):
        zbuf_v[pl.ds(i * 16, 16)] = jnp.zeros((16,), jnp.float32)
        return 0

    lax.fori_loop(0, ROWS_PER_TILE // 16, fill, 0)

    def fill1(i, _):
        ones_v[pl.ds(i * 16, 16)] = jnp.ones((16,), jnp.float32)
        return 0

    lax.fori_loop(0, CH // 16, fill1, 0)

    pltpu.sync_copy(dst_hbm.at[pl.ds(gid * NCHUNK, NCHUNK)], didx)
    pltpu.sync_copy(zbuf_v, cnt_sp.at[pl.ds(s * ROWS_PER_TILE, ROWS_PER_TILE)])
    plsc.subcore_barrier()

    def body(k, _):
        pltpu.sync_copy(ones_v, cnt_sp.at[didx.at[k, 0]], add=True)
        return 0

    lax.fori_loop(0, NCHUNK, body, 0)
    plsc.subcore_barrier()
    pltpu.sync_copy(
        cnt_sp.at[pl.ds(s * ROWS_PER_TILE, ROWS_PER_TILE)],
        out_hbm.at[pl.ds(c * NPAD + s * ROWS_PER_TILE, ROWS_PER_TILE)],
    )


NB = 4
NGRP = NCHUNK // NB


@functools.partial(
    pl.kernel,
    out_type=jax.ShapeDtypeStruct((2 * NPAD, H), jnp.float32),
    mesh=_mesh,
    compiler_params=pltpu.CompilerParams(use_tc_tiling_on_sc=False),
    scratch_types=[
        pltpu.VMEM((NCHUNK, 1, CH), jnp.int32),
        pltpu.VMEM((NCHUNK, 1, CH), jnp.int32),
        pltpu.VMEM((NB, CH, H), jnp.float32),
        pltpu.VMEM_SHARED((NPAD, H), jnp.float32),
        pltpu.SemaphoreType.DMA,
    ],
)
def _agg_kernel(hp_hbm, src_hbm, dst_hbm, out_hbm, sidx, didx, rows, acc_sp, gsem):
    c = lax.axis_index("c")
    s = lax.axis_index("s")
    gid = c * 16 + s

    def fill(t, _):
        rows[0, t // 4, pl.ds((t % 4) * 16, 16)] = jnp.zeros((16,), jnp.float32)
        return 0

    lax.fori_loop(0, CH * (H // 16), fill, 0)

    def zc(k, _):
        pltpu.sync_copy(rows.at[0], acc_sp.at[pl.ds(s * ROWS_PER_TILE + k * CH, CH), :])
        return 0

    lax.fori_loop(0, ROWS_PER_TILE // CH, zc, 0)

    pltpu.sync_copy(src_hbm.at[pl.ds(gid * NCHUNK, NCHUNK)], sidx)
    pltpu.sync_copy(dst_hbm.at[pl.ds(gid * NCHUNK, NCHUNK)], didx)
    plsc.subcore_barrier()

    for b in range(NB):
        pltpu.async_copy(hp_hbm.at[sidx.at[b, 0]], rows.at[b], gsem)

    def grp(q, _):
        k = q * NB
        for b in range(NB):
            pltpu.make_async_copy(hp_hbm.at[sidx.at[k + b, 0]], rows.at[b], gsem).wait()
        for b in range(NB):
            pltpu.sync_copy(rows.at[b], acc_sp.at[didx.at[k + b, 0]], add=True)

            @pl.when(q < NGRP - 1)
            def _(b=b, k=k):
                pltpu.async_copy(hp_hbm.at[sidx.at[k + NB + b, 0]], rows.at[b], gsem)

        return 0

    lax.fori_loop(0, NGRP, grp, 0)
    plsc.subcore_barrier()
    pltpu.sync_copy(
        acc_sp.at[pl.ds(s * ROWS_PER_TILE, ROWS_PER_TILE), :],
        out_hbm.at[pl.ds(c * NPAD + s * ROWS_PER_TILE, ROWS_PER_TILE), :],
    )


def _dense1_body(cnt_ref, x_ref, w1_ref, h_ref, hp_ref):
    cnt2 = cnt_ref[...]
    deg = cnt2[0] + cnt2[1] + 1.0
    dinv = lax.rsqrt(deg)
    h = jnp.dot(x_ref[...], w1_ref[...], preferred_element_type=jnp.float32)
    h_ref[...] = h
    hp_ref[...] = h * dinv


def _dense2_body(parts_ref, h1_ref, cnt_ref, b1_ref, g1_ref, be1_ref, w2_ref,
                 h2_ref, h2p_ref):
    p = parts_ref[...]
    agg = p[0] + p[1]
    cnt2 = cnt_ref[...]
    deg = cnt2[0] + cnt2[1] + 1.0
    dinv = lax.rsqrt(deg)
    z = dinv * agg + (1.0 / deg) * h1_ref[...] + b1_ref[...]
    z = z * (g1_ref[...] / jnp.sqrt(1.0 + EPS)) + be1_ref[...]
    z = jnp.maximum(z, 0.0)
    h2 = jnp.dot(z, w2_ref[...], preferred_element_type=jnp.float32)
    h2_ref[...] = h2
    h2p_ref[...] = h2 * lax.rsqrt(deg)


def _pool_body(parts_ref, h2_ref, cnt_ref, b2_ref, g2_ref, be2_ref, batch_ref,
               out_ref, sums, cnts):
    i = pl.program_id(0)

    @pl.when(i == 0)
    def _():
        sums[...] = jnp.zeros_like(sums)
        cnts[...] = jnp.zeros_like(cnts)

    p = parts_ref[...]
    agg = p[0] + p[1]
    cnt2 = cnt_ref[...]
    deg = cnt2[0] + cnt2[1] + 1.0
    dinv = lax.rsqrt(deg)
    z = dinv * agg + (1.0 / deg) * h2_ref[...] + b2_ref[...]
    z = z * (g2_ref[...] / jnp.sqrt(1.0 + EPS)) + be2_ref[...]
    z = jnp.maximum(z, 0.0)

    b = batch_ref[...]
    onehot = (b == lax.broadcasted_iota(jnp.int32, (BN, G), 1)).astype(jnp.float32)
    sums[...] += lax.dot_general(onehot, z, (((0,), (0,)), ((), ())),
                                 preferred_element_type=jnp.float32)
    cnts[...] += lax.dot_general(onehot, jnp.ones((BN, 1), jnp.float32),
                                 (((0,), (0,)), ((), ())),
                                 preferred_element_type=jnp.float32)
    out_ref[...] = sums[...] / jnp.maximum(cnts[...], 1.0)


def kernel(x, edge_index, batch, W1, b1, g1, beta1, W2, b2, g2, beta2):
    src = edge_index[0].astype(jnp.int32)
    dst = edge_index[1].astype(jnp.int32)
    npad_extra = jnp.arange(EPAD - E, dtype=jnp.int32)
    src_p = jnp.concatenate([src, npad_extra % 256]).reshape(EPAD // CH, 1, CH)
    dst_p = jnp.concatenate([dst, N + npad_extra % (NPAD - N)]).reshape(EPAD // CH, 1, CH)

    cnt_parts = _deg_kernel(dst_p)
    cnt = cnt_parts.reshape(2, NPAD)[:, :N].reshape(2, N, 1)

    cnt_spec = pl.BlockSpec((2, BN, 1), lambda i: (0, i, 0))
    row_spec = pl.BlockSpec((BN, H), lambda i: (i, 0))
    vec_spec = pl.BlockSpec((1, H), lambda i: (0, 0))
    parts_spec = pl.BlockSpec((2, BN, H), lambda i: (0, i, 0))

    h1, h1p = pl.pallas_call(
        _dense1_body,
        grid=(GRID,),
        in_specs=[cnt_spec,
                  pl.BlockSpec((BN, D), lambda i: (i, 0)),
                  pl.BlockSpec((D, H), lambda i: (0, 0))],
        out_specs=[row_spec, row_spec],
        out_shape=[jax.ShapeDtypeStruct((N, H), jnp.float32),
                   jax.ShapeDtypeStruct((N, H), jnp.float32)],
    )(cnt, x, W1)

    agg1 = _agg_kernel(h1p, src_p, dst_p)
    agg1 = agg1.reshape(2, NPAD, H)[:, :N]

    h2, h2p = pl.pallas_call(
        _dense2_body,
        grid=(GRID,),
        in_specs=[parts_spec, row_spec, cnt_spec, vec_spec, vec_spec, vec_spec,
                  pl.BlockSpec((H, H), lambda i: (0, 0))],
        out_specs=[row_spec, row_spec],
        out_shape=[jax.ShapeDtypeStruct((N, H), jnp.float32),
                   jax.ShapeDtypeStruct((N, H), jnp.float32)],
    )(agg1, h1, cnt, b1.reshape(1, H), g1.reshape(1, H), beta1.reshape(1, H), W2)

    agg2 = _agg_kernel(h2p, src_p, dst_p)
    agg2 = agg2.reshape(2, NPAD, H)[:, :N]

    emb = pl.pallas_call(
        _pool_body,
        grid=(GRID,),
        in_specs=[parts_spec, row_spec, cnt_spec, vec_spec, vec_spec, vec_spec,
                  pl.BlockSpec((BN, 1), lambda i: (i, 0))],
        out_specs=pl.BlockSpec((G, H), lambda i: (0, 0)),
        out_shape=jax.ShapeDtypeStruct((G, H), jnp.float32),
        scratch_shapes=[pltpu.VMEM((G, H), jnp.float32),
                        pltpu.VMEM((G, 1), jnp.float32)],
    )(agg2, h2, cnt, b2.reshape(1, H), g2.reshape(1, H), beta2.reshape(1, H),
      batch.astype(jnp.int32).reshape(N, 1))
    return emb


# speedup vs baseline: 39.8787x; 1.0670x over previous
import functools

import jax
import jax.numpy as jnp
from jax import lax
from jax.experimental import pallas as pl
from jax.experimental.pallas import tpu as pltpu
from jax.experimental.pallas import tpu_sc as plsc

N = 10000
E = 320000
D = 128
H = 64
G = 64
EPS = 1e-5

NPAD = 10240
CH = 128
NCHUNKS = E // CH
BASE_CHUNKS = NCHUNKS // 32
EXTRA_TILES = NCHUNKS - 32 * BASE_CHUNKS
ROWS_PER_TILE = NPAD // 16
NB = 6
NGRP = BASE_CHUNKS // NB

BN = 1000
BN2 = 1024
GRID2 = NPAD // BN2

_mesh = plsc.VectorSubcoreMesh(core_axis_name="c", subcore_axis_name="s")


def _tile_range(gid):
    start = gid * BASE_CHUNKS + jnp.minimum(gid, EXTRA_TILES)
    has_extra = gid < EXTRA_TILES
    return start, has_extra


@functools.partial(
    pl.kernel,
    out_type=jax.ShapeDtypeStruct((2 * NPAD,), jnp.float32),
    mesh=_mesh,
    compiler_params=pltpu.CompilerParams(use_tc_tiling_on_sc=False),
    scratch_types=[
        pltpu.VMEM((BASE_CHUNKS + 1, 1, CH), jnp.int32),
        pltpu.VMEM((CH,), jnp.float32),
        pltpu.VMEM((ROWS_PER_TILE,), jnp.float32),
        pltpu.VMEM_SHARED((NPAD,), jnp.float32),
    ],
)
def _deg_kernel(dst_hbm, out_hbm, didx, ones_v, zbuf_v, cnt_sp):
    c = lax.axis_index("c")
    s = lax.axis_index("s")
    gid = c * 16 + s
    start, has_extra = _tile_range(gid)

    def fill(i, _):
        zbuf_v[pl.ds(i * 16, 16)] = jnp.zeros((16,), jnp.float32)
        return 0

    lax.fori_loop(0, ROWS_PER_TILE // 16, fill, 0)

    def fill1(i, _):
        ones_v[pl.ds(i * 16, 16)] = jnp.ones((16,), jnp.float32)
        return 0

    lax.fori_loop(0, CH // 16, fill1, 0)

    pltpu.sync_copy(dst_hbm.at[pl.ds(start, BASE_CHUNKS)],
                    didx.at[pl.ds(0, BASE_CHUNKS)])

    @pl.when(has_extra)
    def _():
        pltpu.sync_copy(dst_hbm.at[pl.ds(start + BASE_CHUNKS, 1)],
                        didx.at[pl.ds(BASE_CHUNKS, 1)])

    pltpu.sync_copy(zbuf_v, cnt_sp.at[pl.ds(s * ROWS_PER_TILE, ROWS_PER_TILE)])
    plsc.subcore_barrier()

    nch = BASE_CHUNKS + has_extra.astype(jnp.int32)

    def body(k, _):
        pltpu.sync_copy(ones_v, cnt_sp.at[didx.at[k, 0]], add=True)
        return 0

    lax.fori_loop(0, nch, body, 0)
    plsc.subcore_barrier()
    pltpu.sync_copy(
        cnt_sp.at[pl.ds(s * ROWS_PER_TILE, ROWS_PER_TILE)],
        out_hbm.at[pl.ds(c * NPAD + s * ROWS_PER_TILE, ROWS_PER_TILE)],
    )


@functools.partial(
    pl.kernel,
    out_type=jax.ShapeDtypeStruct((2 * NPAD, H), jnp.float32),
    mesh=_mesh,
    compiler_params=pltpu.CompilerParams(use_tc_tiling_on_sc=False),
    scratch_types=[
        pltpu.VMEM((BASE_CHUNKS + 1, 1, CH), jnp.int32),
        pltpu.VMEM((BASE_CHUNKS + 1, 1, CH), jnp.int32),
        pltpu.VMEM((NB, CH, H), jnp.float32),
        pltpu.VMEM_SHARED((NPAD, H), jnp.float32),
        pltpu.SemaphoreType.DMA,
    ],
)
def _agg_kernel(hp_hbm, src_hbm, dst_hbm, out_hbm, sidx, didx, rows, acc_sp, gsem):
    c = lax.axis_index("c")
    s = lax.axis_index("s")
    gid = c * 16 + s
    start, has_extra = _tile_range(gid)

    def fill(t, _):
        rows[0, t // 4, pl.ds((t % 4) * 16, 16)] = jnp.zeros((16,), jnp.float32)
        return 0

    lax.fori_loop(0, CH * (H // 16), fill, 0)

    def zc(k, _):
        pltpu.sync_copy(rows.at[0], acc_sp.at[pl.ds(s * ROWS_PER_TILE + k * CH, CH), :])
        return 0

    lax.fori_loop(0, ROWS_PER_TILE // CH, zc, 0)

    pltpu.sync_copy(src_hbm.at[pl.ds(start, BASE_CHUNKS)],
                    sidx.at[pl.ds(0, BASE_CHUNKS)])
    pltpu.sync_copy(dst_hbm.at[pl.ds(start, BASE_CHUNKS)],
                    didx.at[pl.ds(0, BASE_CHUNKS)])

    @pl.when(has_extra)
    def _():
        pltpu.sync_copy(src_hbm.at[pl.ds(start + BASE_CHUNKS, 1)],
                        sidx.at[pl.ds(BASE_CHUNKS, 1)])
        pltpu.sync_copy(dst_hbm.at[pl.ds(start + BASE_CHUNKS, 1)],
                        didx.at[pl.ds(BASE_CHUNKS, 1)])

    plsc.subcore_barrier()

    for b in range(NB):
        pltpu.async_copy(hp_hbm.at[sidx.at[b, 0]], rows.at[b], gsem)

    def grp(q, _):
        k = q * NB
        for b in range(NB):
            pltpu.make_async_copy(hp_hbm.at[sidx.at[k + b, 0]], rows.at[b], gsem).wait()
        for b in range(NB):
            pltpu.sync_copy(rows.at[b], acc_sp.at[didx.at[k + b, 0]], add=True)

            @pl.when(q < NGRP - 1)
            def _(b=b, k=k):
                pltpu.async_copy(hp_hbm.at[sidx.at[k + NB + b, 0]], rows.at[b], gsem)

        return 0

    lax.fori_loop(0, NGRP, grp, 0)

    @pl.when(has_extra)
    def _():
        pltpu.async_copy(hp_hbm.at[sidx.at[BASE_CHUNKS, 0]], rows.at[0], gsem).wait()
        pltpu.sync_copy(rows.at[0], acc_sp.at[didx.at[BASE_CHUNKS, 0]], add=True)

    plsc.subcore_barrier()
    pltpu.sync_copy(
        acc_sp.at[pl.ds(s * ROWS_PER_TILE, ROWS_PER_TILE), :],
        out_hbm.at[pl.ds(c * NPAD + s * ROWS_PER_TILE, ROWS_PER_TILE), :],
    )


def _mm1_body(x_ref, w1_ref, h_ref):
    h_ref[...] = jnp.dot(x_ref[...], w1_ref[...],
                         preferred_element_type=jnp.float32)


def _scale1_body(cnt_ref, h_ref, hp_ref):
    cnt2 = cnt_ref[...]
    deg = cnt2[0] + cnt2[1] + 1.0
    hp_ref[...] = h_ref[...] * lax.rsqrt(deg)


def _dense2_body(parts_ref, h1_ref, cnt_ref, b1_ref, g1_ref, be1_ref, w2_ref,
                 h2_ref, h2p_ref):
    p = parts_ref[...]
    agg = p[0] + p[1]
    cnt2 = cnt_ref[...]
    deg = cnt2[0] + cnt2[1] + 1.0
    dinv = lax.rsqrt(deg)
    z = dinv * agg + (1.0 / deg) * h1_ref[...] + b1_ref[...]
    z = z * (g1_ref[...] / jnp.sqrt(1.0 + EPS)) + be1_ref[...]
    z = jnp.maximum(z, 0.0)
    h2 = jnp.dot(z, w2_ref[...], preferred_element_type=jnp.float32)
    h2_ref[...] = h2
    h2p_ref[...] = h2 * dinv


def _pool_body(parts_ref, h2_ref, cnt_ref, b2_ref, g2_ref, be2_ref, batch_ref,
               out_ref, sums, cnts):
    i = pl.program_id(0)

    @pl.when(i == 0)
    def _():
        sums[...] = jnp.zeros_like(sums)
        cnts[...] = jnp.zeros_like(cnts)

    p = parts_ref[...]
    agg = p[0] + p[1]
    cnt2 = cnt_ref[...]
    deg = cnt2[0] + cnt2[1] + 1.0
    dinv = lax.rsqrt(deg)
    z = dinv * agg + (1.0 / deg) * h2_ref[...] + b2_ref[...]
    z = z * (g2_ref[...] / jnp.sqrt(1.0 + EPS)) + be2_ref[...]
    z = jnp.maximum(z, 0.0)

    b = batch_ref[...]
    z = jnp.where(b < G, z, 0.0)
    onehot = (b == lax.broadcasted_iota(jnp.int32, (BN2, G), 1)).astype(jnp.float32)
    sums[...] += lax.dot_general(onehot, z, (((0,), (0,)), ((), ())),
                                 preferred_element_type=jnp.float32)
    cnts[...] += lax.dot_general(onehot, jnp.ones((BN2, 1), jnp.float32),
                                 (((0,), (0,)), ((), ())),
                                 preferred_element_type=jnp.float32)
    out_ref[...] = sums[...] / jnp.maximum(cnts[...], 1.0)


def kernel(x, edge_index, batch, W1, b1, g1, beta1, W2, b2, g2, beta2):
    src3 = edge_index[0].astype(jnp.int32).reshape(NCHUNKS, 1, CH)
    dst3 = edge_index[1].astype(jnp.int32).reshape(NCHUNKS, 1, CH)
    batch_p = jnp.pad(batch.astype(jnp.int32), (0, NPAD - N),
                      constant_values=G).reshape(NPAD, 1)

    cnt = _deg_kernel(dst3).reshape(2, NPAD, 1)

    h1 = pl.pallas_call(
        _mm1_body,
        grid=(N // BN,),
        in_specs=[pl.BlockSpec((BN, D), lambda i: (i, 0)),
                  pl.BlockSpec((D, H), lambda i: (0, 0))],
        out_specs=pl.BlockSpec((BN, H), lambda i: (i, 0)),
        out_shape=jax.ShapeDtypeStruct((NPAD, H), jnp.float32),
    )(x, W1)

    cnt_spec = pl.BlockSpec((2, BN2, 1), lambda i: (0, i, 0))
    row_spec = pl.BlockSpec((BN2, H), lambda i: (i, 0))
    vec_spec = pl.BlockSpec((1, H), lambda i: (0, 0))
    parts_spec = pl.BlockSpec((2, BN2, H), lambda i: (0, i, 0))

    h1p = pl.pallas_call(
        _scale1_body,
        grid=(GRID2,),
        in_specs=[cnt_spec, row_spec],
        out_specs=row_spec,
        out_shape=jax.ShapeDtypeStruct((NPAD, H), jnp.float32),
    )(cnt, h1)

    agg1 = _agg_kernel(h1p, src3, dst3).reshape(2, NPAD, H)

    h2, h2p = pl.pallas_call(
        _dense2_body,
        grid=(GRID2,),
        in_specs=[parts_spec, row_spec, cnt_spec, vec_spec, vec_spec, vec_spec,
                  pl.BlockSpec((H, H), lambda i: (0, 0))],
        out_specs=[row_spec, row_spec],
        out_shape=[jax.ShapeDtypeStruct((NPAD, H), jnp.float32),
                   jax.ShapeDtypeStruct((NPAD, H), jnp.float32)],
    )(agg1, h1, cnt, b1.reshape(1, H), g1.reshape(1, H), beta1.reshape(1, H), W2)

    agg2 = _agg_kernel(h2p, src3, dst3).reshape(2, NPAD, H)

    emb = pl.pallas_call(
        _pool_body,
        grid=(GRID2,),
        in_specs=[parts_spec, row_spec, cnt_spec, vec_spec, vec_spec, vec_spec,
                  pl.BlockSpec((BN2, 1), lambda i: (i, 0))],
        out_specs=pl.BlockSpec((G, H), lambda i: (0, 0)),
        out_shape=jax.ShapeDtypeStruct((G, H), jnp.float32),
        scratch_shapes=[pltpu.VMEM((G, H), jnp.float32),
                        pltpu.VMEM((G, 1), jnp.float32)],
    )(agg2, h2, cnt, b2.reshape(1, H), g2.reshape(1, H), beta2.reshape(1, H),
      batch_p)
    return emb


# speedup vs baseline: 40.3414x; 1.0116x over previous
import functools

import jax
import jax.numpy as jnp
from jax import lax
from jax.experimental import pallas as pl
from jax.experimental.pallas import tpu as pltpu
from jax.experimental.pallas import tpu_sc as plsc

N = 10000
E = 320000
D = 128
H = 64
G = 64
EPS = 1e-5

NPAD = 10240
CH = 128
NCHUNKS = E // CH
BASE_CHUNKS = NCHUNKS // 32
EXTRA_TILES = NCHUNKS - 32 * BASE_CHUNKS
ROWS_PER_TILE = NPAD // 16
NB = 6
NGRP = BASE_CHUNKS // NB

BN = 1000
BN2 = 1024
GRID2 = NPAD // BN2

_mesh = plsc.VectorSubcoreMesh(core_axis_name="c", subcore_axis_name="s")


def _tile_range(gid):
    start = gid * BASE_CHUNKS + jnp.minimum(gid, EXTRA_TILES)
    has_extra = gid < EXTRA_TILES
    return start, has_extra


@functools.partial(
    pl.kernel,
    out_type=jax.ShapeDtypeStruct((2 * NPAD,), jnp.float32),
    mesh=_mesh,
    compiler_params=pltpu.CompilerParams(use_tc_tiling_on_sc=False),
    scratch_types=[
        pltpu.VMEM((BASE_CHUNKS + 1, 1, CH), jnp.int32),
        pltpu.VMEM((CH,), jnp.float32),
        pltpu.VMEM((ROWS_PER_TILE,), jnp.float32),
        pltpu.VMEM_SHARED((NPAD,), jnp.float32),
    ],
)
def _deg_kernel(dst_hbm, out_hbm, didx, ones_v, zbuf_v, cnt_sp):
    c = lax.axis_index("c")
    s = lax.axis_index("s")
    gid = c * 16 + s
    start, has_extra = _tile_range(gid)

    def fill(i, _):
        zbuf_v[pl.ds(i * 16, 16)] = jnp.zeros((16,), jnp.float32)
        return 0

    lax.fori_loop(0, ROWS_PER_TILE // 16, fill, 0)

    def fill1(i, _):
        ones_v[pl.ds(i * 16, 16)] = jnp.ones((16,), jnp.float32)
        return 0

    lax.fori_loop(0, CH // 16, fill1, 0)

    pltpu.sync_copy(dst_hbm.at[pl.ds(start, BASE_CHUNKS)],
                    didx.at[pl.ds(0, BASE_CHUNKS)])

    @pl.when(has_extra)
    def _():
        pltpu.sync_copy(dst_hbm.at[pl.ds(start + BASE_CHUNKS, 1)],
                        didx.at[pl.ds(BASE_CHUNKS, 1)])

    pltpu.sync_copy(zbuf_v, cnt_sp.at[pl.ds(s * ROWS_PER_TILE, ROWS_PER_TILE)])
    plsc.subcore_barrier()

    nch = BASE_CHUNKS + has_extra.astype(jnp.int32)

    def body(k, _):
        pltpu.sync_copy(ones_v, cnt_sp.at[didx.at[k, 0]], add=True)
        return 0

    lax.fori_loop(0, nch, body, 0)
    plsc.subcore_barrier()
    pltpu.sync_copy(
        cnt_sp.at[pl.ds(s * ROWS_PER_TILE, ROWS_PER_TILE)],
        out_hbm.at[pl.ds(c * NPAD + s * ROWS_PER_TILE, ROWS_PER_TILE)],
    )


@functools.partial(
    pl.kernel,
    out_type=jax.ShapeDtypeStruct((2 * NPAD, H), jnp.float32),
    mesh=_mesh,
    compiler_params=pltpu.CompilerParams(use_tc_tiling_on_sc=False),
    scratch_types=[
        pltpu.VMEM((BASE_CHUNKS + 1, 1, CH), jnp.int32),
        pltpu.VMEM((BASE_CHUNKS + 1, 1, CH), jnp.int32),
        pltpu.VMEM((NB, CH, H), jnp.float32),
        pltpu.VMEM_SHARED((NPAD, H), jnp.float32),
        pltpu.SemaphoreType.DMA,
    ],
)
def _agg_kernel(hp_hbm, src_hbm, dst_hbm, out_hbm, sidx, didx, rows, acc_sp, gsem):
    c = lax.axis_index("c")
    s = lax.axis_index("s")
    gid = c * 16 + s
    start, has_extra = _tile_range(gid)

    def fill(t, _):
        rows[0, t // 4, pl.ds((t % 4) * 16, 16)] = jnp.zeros((16,), jnp.float32)
        return 0

    lax.fori_loop(0, CH * (H // 16), fill, 0)

    def zc(k, _):
        pltpu.sync_copy(rows.at[0], acc_sp.at[pl.ds(s * ROWS_PER_TILE + k * CH, CH), :])
        return 0

    lax.fori_loop(0, ROWS_PER_TILE // CH, zc, 0)

    pltpu.sync_copy(src_hbm.at[pl.ds(start, BASE_CHUNKS)],
                    sidx.at[pl.ds(0, BASE_CHUNKS)])
    pltpu.sync_copy(dst_hbm.at[pl.ds(start, BASE_CHUNKS)],
                    didx.at[pl.ds(0, BASE_CHUNKS)])

    @pl.when(has_extra)
    def _():
        pltpu.sync_copy(src_hbm.at[pl.ds(start + BASE_CHUNKS, 1)],
                        sidx.at[pl.ds(BASE_CHUNKS, 1)])
        pltpu.sync_copy(dst_hbm.at[pl.ds(start + BASE_CHUNKS, 1)],
                        didx.at[pl.ds(BASE_CHUNKS, 1)])

    plsc.subcore_barrier()

    for b in range(NB):
        pltpu.async_copy(hp_hbm.at[sidx.at[b, 0]], rows.at[b], gsem)

    def grp(q, _):
        k = q * NB
        for b in range(NB):
            pltpu.make_async_copy(hp_hbm.at[sidx.at[k + b, 0]], rows.at[b], gsem).wait()
        for b in range(NB):
            pltpu.sync_copy(rows.at[b], acc_sp.at[didx.at[k + b, 0]], add=True)

            @pl.when(q < NGRP - 1)
            def _(b=b, k=k):
                pltpu.async_copy(hp_hbm.at[sidx.at[k + NB + b, 0]], rows.at[b], gsem)

        return 0

    lax.fori_loop(0, NGRP, grp, 0)

    @pl.when(has_extra)
    def _():
        pltpu.async_copy(hp_hbm.at[sidx.at[BASE_CHUNKS, 0]], rows.at[0], gsem).wait()
        pltpu.sync_copy(rows.at[0], acc_sp.at[didx.at[BASE_CHUNKS, 0]], add=True)

    plsc.subcore_barrier()
    pltpu.sync_copy(
        acc_sp.at[pl.ds(s * ROWS_PER_TILE, ROWS_PER_TILE), :],
        out_hbm.at[pl.ds(c * NPAD + s * ROWS_PER_TILE, ROWS_PER_TILE), :],
    )


def _mm1_body(cnt_ref, x_ref, w1_ref, hp_ref):
    cnt2 = cnt_ref[...]
    deg = cnt2[0] + cnt2[1] + 1.0
    hp_ref[...] = jnp.dot(x_ref[...], w1_ref[...],
                          preferred_element_type=jnp.float32) * lax.rsqrt(deg)


def _dense2_body(parts_ref, h1p_ref, cnt_ref, b1_ref, g1_ref, be1_ref, w2_ref,
                 h2p_ref):
    p = parts_ref[...]
    agg = p[0] + p[1]
    cnt2 = cnt_ref[...]
    deg = cnt2[0] + cnt2[1] + 1.0
    dinv = lax.rsqrt(deg)
    z = dinv * (agg + h1p_ref[...]) + b1_ref[...]
    z = z * (g1_ref[...] / jnp.sqrt(1.0 + EPS)) + be1_ref[...]
    z = jnp.maximum(z, 0.0)
    h2p_ref[...] = jnp.dot(z, w2_ref[...],
                           preferred_element_type=jnp.float32) * dinv


def _pool_body(parts_ref, h2p_ref, cnt_ref, b2_ref, g2_ref, be2_ref, batch_ref,
               out_ref, sums, cnts):
    i = pl.program_id(0)

    @pl.when(i == 0)
    def _():
        sums[...] = jnp.zeros_like(sums)
        cnts[...] = jnp.zeros_like(cnts)

    p = parts_ref[...]
    agg = p[0] + p[1]
    cnt2 = cnt_ref[...]
    deg = cnt2[0] + cnt2[1] + 1.0
    dinv = lax.rsqrt(deg)
    z = dinv * (agg + h2p_ref[...]) + b2_ref[...]
    z = z * (g2_ref[...] / jnp.sqrt(1.0 + EPS)) + be2_ref[...]
    z = jnp.maximum(z, 0.0)

    b = batch_ref[...]
    z = jnp.where(b < G, z, 0.0)
    onehot = (b == lax.broadcasted_iota(jnp.int32, (BN2, G), 1)).astype(jnp.float32)
    sums[...] += lax.dot_general(onehot, z, (((0,), (0,)), ((), ())),
                                 preferred_element_type=jnp.float32)
    cnts[...] += lax.dot_general(onehot, jnp.ones((BN2, 1), jnp.float32),
                                 (((0,), (0,)), ((), ())),
                                 preferred_element_type=jnp.float32)
    out_ref[...] = sums[...] / jnp.maximum(cnts[...], 1.0)


def kernel(x, edge_index, batch, W1, b1, g1, beta1, W2, b2, g2, beta2):
    src3 = edge_index[0].astype(jnp.int32).reshape(NCHUNKS, 1, CH)
    dst3 = edge_index[1].astype(jnp.int32).reshape(NCHUNKS, 1, CH)
    batch_p = jnp.pad(batch.astype(jnp.int32), (0, NPAD - N),
                      constant_values=G).reshape(NPAD, 1)

    cnt = _deg_kernel(dst3).reshape(2, NPAD, 1)

    h1p = pl.pallas_call(
        _mm1_body,
        grid=(N // BN,),
        in_specs=[pl.BlockSpec((2, BN, 1), lambda i: (0, i, 0)),
                  pl.BlockSpec((BN, D), lambda i: (i, 0)),
                  pl.BlockSpec((D, H), lambda i: (0, 0))],
        out_specs=pl.BlockSpec((BN, H), lambda i: (i, 0)),
        out_shape=jax.ShapeDtypeStruct((NPAD, H), jnp.float32),
    )(cnt, x, W1)

    cnt_spec = pl.BlockSpec((2, BN2, 1), lambda i: (0, i, 0))
    row_spec = pl.BlockSpec((BN2, H), lambda i: (i, 0))
    vec_spec = pl.BlockSpec((1, H), lambda i: (0, 0))
    parts_spec = pl.BlockSpec((2, BN2, H), lambda i: (0, i, 0))

    agg1 = _agg_kernel(h1p, src3, dst3).reshape(2, NPAD, H)

    h2p = pl.pallas_call(
        _dense2_body,
        grid=(GRID2,),
        in_specs=[parts_spec, row_spec, cnt_spec, vec_spec, vec_spec, vec_spec,
                  pl.BlockSpec((H, H), lambda i: (0, 0))],
        out_specs=row_spec,
        out_shape=jax.ShapeDtypeStruct((NPAD, H), jnp.float32),
    )(agg1, h1p, cnt, b1.reshape(1, H), g1.reshape(1, H), beta1.reshape(1, H),
      W2)

    agg2 = _agg_kernel(h2p, src3, dst3).reshape(2, NPAD, H)

    emb = pl.pallas_call(
        _pool_body,
        grid=(GRID2,),
        in_specs=[parts_spec, row_spec, cnt_spec, vec_spec, vec_spec, vec_spec,
                  pl.BlockSpec((BN2, 1), lambda i: (i, 0))],
        out_specs=pl.BlockSpec((G, H), lambda i: (0, 0)),
        out_shape=jax.ShapeDtypeStruct((G, H), jnp.float32),
        scratch_shapes=[pltpu.VMEM((G, H), jnp.float32),
                        pltpu.VMEM((G, 1), jnp.float32)],
    )(agg2, h2p, cnt, b2.reshape(1, H), g2.reshape(1, H), beta2.reshape(1, H),
      batch_p)
    return emb
